# baseline (speedup 1.0000x reference)
import functools
import math

import jax
import jax.numpy as jnp
import numpy as np
from jax import lax
from jax.experimental import pallas as pl
from jax.experimental.pallas import tpu as pltpu
from jax.experimental.pallas import tpu_sc as plsc

D = 128
T_DIM = 128
NGH = 80
N_HEAD = 4
DH = D // N_HEAD
NUM_NTYPE = 3
NUM_ETYPE = 3

_BS = 32

_NW = 32
_CHUNK = 128


def _sc_gather(node_table, memory, edge_table, node_idx, seed_idx, edge_idx):
    NR = edge_idx.shape[0]
    NS = seed_idx.shape[0]
    per_w = NR // _NW
    n_chunks = per_w // _CHUNK
    seed_per_w = NS // _NW

    mesh = plsc.VectorSubcoreMesh(core_axis_name="c", subcore_axis_name="s")

    def _vadd_into(dst, src, rows):
        def rowfn(r, carry):
            for c in range(D // 16):
                sl = pl.ds(c * 16, 16)
                dst[r, sl] = dst[r, sl] + src[r, sl]
            return carry
        lax.fori_loop(0, rows, rowfn, 0)

    @functools.partial(
        pl.kernel,
        out_type=(
            jax.ShapeDtypeStruct((NR, D), jnp.float32),
            jax.ShapeDtypeStruct((NR, D), jnp.float32),
            jax.ShapeDtypeStruct((NS, D), jnp.float32),
        ),
        mesh=mesh,
        scratch_types=[
            pltpu.VMEM((_CHUNK,), jnp.int32),
            pltpu.VMEM((_CHUNK,), jnp.int32),
            pltpu.VMEM((_CHUNK, D), jnp.float32),
            pltpu.VMEM((_CHUNK, D), jnp.float32),
            pltpu.VMEM((_CHUNK, D), jnp.float32),
            pltpu.VMEM((seed_per_w,), jnp.int32),
            pltpu.VMEM((seed_per_w, D), jnp.float32),
            pltpu.VMEM((seed_per_w, D), jnp.float32),
            pltpu.SemaphoreType.DMA,
            pltpu.SemaphoreType.DMA,
            pltpu.SemaphoreType.DMA,
        ],
    )
    def k(ntab, mtab, etab, nidx, sidx, eidx, out_f, out_e, out_sf,
          nidx_v, eidx_v, nbuf, mbuf, ebuf, sidx_v, snbuf, smbuf,
          sem_n, sem_m, sem_e):
        wid = lax.axis_index("s") * 2 + lax.axis_index("c")
        base0 = wid * per_w

        def body(i, carry):
            base = base0 + i * _CHUNK
            pltpu.sync_copy(nidx.at[pl.ds(base, _CHUNK)], nidx_v)
            pltpu.sync_copy(eidx.at[pl.ds(base, _CHUNK)], eidx_v)
            cn = pltpu.async_copy(ntab.at[nidx_v], nbuf, sem_n)
            cm = pltpu.async_copy(mtab.at[nidx_v], mbuf, sem_m)
            ce = pltpu.async_copy(etab.at[eidx_v], ebuf, sem_e)
            ce.wait()
            pltpu.sync_copy(ebuf, out_e.at[pl.ds(base, _CHUNK)])
            cn.wait()
            cm.wait()
            _vadd_into(nbuf, mbuf, _CHUNK)
            pltpu.sync_copy(nbuf, out_f.at[pl.ds(base, _CHUNK)])
            return carry

        lax.fori_loop(0, n_chunks, body, 0)

        sbase = wid * seed_per_w
        pltpu.sync_copy(sidx.at[pl.ds(sbase, seed_per_w)], sidx_v)
        cn = pltpu.async_copy(ntab.at[sidx_v], snbuf, sem_n)
        cm = pltpu.async_copy(mtab.at[sidx_v], smbuf, sem_m)
        cn.wait()
        cm.wait()
        _vadd_into(snbuf, smbuf, seed_per_w)
        pltpu.sync_copy(snbuf, out_sf.at[pl.ds(sbase, seed_per_w)])

    return k(node_table, memory, edge_table, node_idx, seed_idx, edge_idx)


_COSC = (1.0, -19.73920440673828, 64.93911743164062, -85.45014190673828,
         60.16762924194336, -25.967592239379883, 6.52864933013916)
_MAGIC = 1.5 * (2.0 ** 23)
_INV2PI = 0.15915494309189535


def _cos2pi(r):
    k = (r + _MAGIC) - _MAGIC
    f = r - k
    u = f * f
    acc = u * _COSC[6] + _COSC[5]
    for c in _COSC[4::-1]:
        acc = acc * u + c
    return acc


def _typed(x2, oh2, Wn):
    acc = None
    for t in range(NUM_NTYPE):
        y = jnp.dot(x2, Wn[t], preferred_element_type=jnp.float32)
        y = y * oh2[:, t : t + 1]
        acc = y if acc is None else acc + y
    return acc


def _bf(x):
    return x.astype(jnp.bfloat16)


def _fwd_kernel(
    src_s, src_t, nf_s, nf_t,
    ne_s, ne_t, dt_s, dt_t, pen_s, pen_t,
    vm_s, vm_t, nm_s, nm_t, em,
    freq, phase, Wn, Wq, Wk, Wv, f1w, f1b, f2w, f2b, Wd, Hm, Hmt,
    out_ref,
):
    freq_r = freq[...] * _INV2PI
    phase_r = phase[...] * _INV2PI
    Wn_a = Wn[...]
    Wk_a = Wk[...]
    Wv_a = Wv[...]
    Wq_a = Wq[...]
    f1w_a = f1w[...]
    Hm_a = Hm[...]
    Hmt_a = Hmt[...]
    qt = jnp.dot(jnp.cos(phase[...]), Wq_a[D:], preferred_element_type=jnp.float32)

    def local_half(src_ref, nf_ref, ne_ref, dt_ref,
                   pen_ref, vm_ref, nm_ref):
        BS = src_ref.shape[0]
        R = BS * NGH
        feat2 = _bf(nf_ref[...]).reshape(R, D)
        edge2 = _bf(ne_ref[...]).reshape(R, D)
        targ = jnp.dot(dt_ref[...], freq_r, preferred_element_type=jnp.float32)
        temb2 = _bf(_cos2pi(targ + phase_r))
        vm2 = vm_ref[...].reshape(R, NUM_NTYPE)

        trans2 = _bf(_typed(feat2, vm2, _bf(Wn_a)))
        Wk_b = _bf(Wk_a)
        Wv_b = _bf(Wv_a)
        k2 = (
            jnp.dot(trans2, Wk_b[:D], preferred_element_type=jnp.float32)
            + jnp.dot(edge2, Wk_b[D : 2 * D], preferred_element_type=jnp.float32)
            + jnp.dot(temb2, Wk_b[2 * D :], preferred_element_type=jnp.float32)
        )
        v2 = (
            jnp.dot(trans2, Wv_b[:D], preferred_element_type=jnp.float32)
            + jnp.dot(edge2, Wv_b[D : 2 * D], preferred_element_type=jnp.float32)
            + jnp.dot(temb2, Wv_b[2 * D :], preferred_element_type=jnp.float32)
        )

        src_f = src_ref[...]
        trans_s = _typed(src_f, nm_ref[...], Wn_a)
        q = jnp.dot(trans_s, Wq_a[:D], preferred_element_type=jnp.float32) + qt

        q3 = jax.lax.broadcast_in_dim(q, (BS, NGH, D), (0, 2))
        qp3 = k2.reshape(BS, NGH, D) * q3
        s2 = jnp.dot(qp3.reshape(R, D), Hm_a, preferred_element_type=jnp.float32)
        s3 = s2.reshape(BS, NGH, 8) + pen_ref[...].reshape(BS, NGH, 1)
        e = jnp.exp(s3)
        z = jnp.sum(e, axis=1, keepdims=True)
        a3 = e / z
        A2 = jnp.dot(a3.reshape(R, 8), Hmt_a, preferred_element_type=jnp.float32)
        w3 = (A2 * v2).reshape(BS, NGH, D)
        ao = jnp.sum(w3, axis=1)
        h1 = jax.nn.relu(
            jnp.dot(ao, f1w_a[:D], preferred_element_type=jnp.float32)
            + jnp.dot(src_f, f1w_a[D:], preferred_element_type=jnp.float32)
            + f1b[...]
        )
        return jnp.dot(h1, f2w[...], preferred_element_type=jnp.float32) + f2b[...]

    local_s = local_half(src_s, nf_s, ne_s, dt_s, pen_s, vm_s, nm_s)
    local_t = local_half(src_t, nf_t, ne_t, dt_t, pen_t, vm_t, nm_t)

    proj = _typed(local_t, em[...], Wd[...])
    s = jnp.sum(local_s * proj, axis=1, keepdims=True)
    out_ref[0] = 1.0 / (1.0 + jnp.exp(-s))


def _tc_forward(src_feat, ngh_feat, ngh_edge, dt, pen, vtype_oh,
                ntype_oh, etype_oh, basis_freq, phase, W_ntype, Wq, Wk, Wv,
                fc1_w, fc1_b, fc2_w, fc2_b, W_dec):
    B = etype_oh.shape[0]
    BS = _BS
    G = B // BS

    hm = np.zeros((D, 8), np.float32)
    for h in range(N_HEAD):
        hm[h * DH : (h + 1) * DH, h] = 1.0
    Hm = jnp.asarray(hm / math.sqrt(DH))
    Hmt = jnp.asarray(hm.T.copy())

    def half_spec(shape, off):
        nd = len(shape)
        blk = (shape[0] // (2 * G),) + shape[1:]
        return pl.BlockSpec(blk, lambda i, o=off: (o + i,) + (0,) * (nd - 1))

    def full_spec(shape):
        nd = len(shape)
        return pl.BlockSpec(shape, lambda i: (0,) * nd)

    in_specs = []
    args = []

    def add_pair(x):
        args.extend([x, x])
        in_specs.extend([half_spec(x.shape, 0), half_spec(x.shape, G)])

    add_pair(src_feat)
    add_pair(ngh_feat)
    add_pair(ngh_edge)
    add_pair(dt)
    add_pair(pen)
    add_pair(vtype_oh)
    add_pair(ntype_oh)
    args.append(etype_oh)
    in_specs.append(pl.BlockSpec((BS, NUM_ETYPE), lambda i: (i, 0)))

    for w in (basis_freq.reshape(1, T_DIM), phase.reshape(1, T_DIM), W_ntype,
              Wq, Wk, Wv, fc1_w, fc1_b.reshape(1, D), fc2_w,
              fc2_b.reshape(1, D), W_dec, Hm, Hmt):
        args.append(w)
        in_specs.append(full_spec(w.shape))

    out = pl.pallas_call(
        _fwd_kernel,
        grid=(G,),
        in_specs=in_specs,
        out_specs=pl.BlockSpec((1, BS, 1), lambda i: (i, 0, 0)),
        out_shape=jax.ShapeDtypeStruct((G, BS, 1), jnp.float32),
    )(*args)
    return out.reshape(B)


def kernel(src_idx_l, tgt_idx_l, cut_time_l, src_utype_l, tgt_utype_l, etype_l,
           ngh_node, ngh_eidx, ngh_t, ngh_etype, ngh_vtype,
           node_table, edge_table, memory, basis_freq, phase,
           W_ntype, Wq, Wk, Wv, fc1_w, fc1_b, fc2_w, fc2_b, W_dec):
    n = src_idx_l.shape[0]
    nodes = jnp.concatenate([src_idx_l, tgt_idx_l])
    times = jnp.concatenate([cut_time_l, cut_time_l])
    ntypes = jnp.concatenate([src_utype_l, tgt_utype_l])

    dt = (times[:, None] - ngh_t).reshape(-1, 1)
    pen = jnp.where(ngh_node == 0, -30.0, 0.0).astype(jnp.float32).reshape(-1, 1)
    vtype_oh = jax.nn.one_hot(ngh_vtype, NUM_NTYPE, dtype=jnp.bfloat16)
    ntype_oh = jax.nn.one_hot(ntypes, NUM_NTYPE, dtype=jnp.float32)
    etype_oh = jax.nn.one_hot(etype_l, NUM_ETYPE, dtype=jnp.float32)

    out_f, out_e, out_sf = _sc_gather(
        node_table, memory, edge_table,
        ngh_node.reshape(-1).astype(jnp.int32),
        nodes.astype(jnp.int32),
        ngh_eidx.reshape(-1).astype(jnp.int32))
    n2b = 2 * n
    return _tc_forward(
        out_sf,
        out_f.reshape(n2b, NGH, D), out_e.reshape(n2b, NGH, D),
        dt, pen, vtype_oh, ntype_oh, etype_oh, basis_freq, phase,
        W_ntype, Wq, Wk, Wv, fc1_w, fc1_b, fc2_w, fc2_b, W_dec)


# speedup vs baseline: 1.0979x; 1.0979x over previous
import functools
import math

import jax
import jax.numpy as jnp
import numpy as np
from jax import lax
from jax.experimental import pallas as pl
from jax.experimental.pallas import tpu as pltpu
from jax.experimental.pallas import tpu_sc as plsc

D = 128
T_DIM = 128
NGH = 80
N_HEAD = 4
DH = D // N_HEAD
NUM_NTYPE = 3
NUM_ETYPE = 3

_BS = 32

_NW = 32
_CHUNK = 128


def _sc_gather(node_table, memory, edge_table, node_idx, seed_idx, edge_idx):
    NR = edge_idx.shape[0]
    NS = seed_idx.shape[0]
    per_w = NR // _NW
    n_chunks = per_w // _CHUNK
    seed_per_w = NS // _NW

    mesh = plsc.VectorSubcoreMesh(core_axis_name="c", subcore_axis_name="s")

    def _vadd_into(dst, src, rows):
        def rowfn(r, carry):
            for c in range(D // 16):
                sl = pl.ds(c * 16, 16)
                dst[r, sl] = dst[r, sl] + src[r, sl]
            return carry
        lax.fori_loop(0, rows, rowfn, 0)

    @functools.partial(
        pl.kernel,
        out_type=(
            jax.ShapeDtypeStruct((NR, D), jnp.float32),
            jax.ShapeDtypeStruct((NR, D), jnp.float32),
            jax.ShapeDtypeStruct((NS, D), jnp.float32),
        ),
        mesh=mesh,
        scratch_types=[
            pltpu.VMEM((_CHUNK,), jnp.int32),
            pltpu.VMEM((_CHUNK,), jnp.int32),
            pltpu.VMEM((_CHUNK,), jnp.int32),
            pltpu.VMEM((_CHUNK,), jnp.int32),
            pltpu.VMEM((_CHUNK, D), jnp.float32),
            pltpu.VMEM((_CHUNK, D), jnp.float32),
            pltpu.VMEM((_CHUNK, D), jnp.float32),
            pltpu.VMEM((_CHUNK, D), jnp.float32),
            pltpu.VMEM((_CHUNK, D), jnp.float32),
            pltpu.VMEM((_CHUNK, D), jnp.float32),
            pltpu.VMEM((seed_per_w,), jnp.int32),
            pltpu.VMEM((seed_per_w, D), jnp.float32),
            pltpu.VMEM((seed_per_w, D), jnp.float32),
            pltpu.SemaphoreType.DMA,
            pltpu.SemaphoreType.DMA,
            pltpu.SemaphoreType.DMA,
            pltpu.SemaphoreType.DMA,
            pltpu.SemaphoreType.DMA,
            pltpu.SemaphoreType.DMA,
        ],
    )
    def k(ntab, mtab, etab, nidx, sidx, eidx, out_f, out_e, out_sf,
          nidx_a, eidx_a, nidx_b, eidx_b,
          nbuf_a, mbuf_a, ebuf_a, nbuf_b, mbuf_b, ebuf_b,
          sidx_v, snbuf, smbuf,
          sem_na, sem_ma, sem_ea, sem_nb, sem_mb, sem_eb):
        wid = lax.axis_index("s") * 2 + lax.axis_index("c")
        base0 = wid * per_w
        sets = (
            (nidx_a, eidx_a, nbuf_a, mbuf_a, ebuf_a, sem_na, sem_ma, sem_ea),
            (nidx_b, eidx_b, nbuf_b, mbuf_b, ebuf_b, sem_nb, sem_mb, sem_eb),
        )

        def fire(i, st):
            nv, ev, nb, mb, eb, sn, sm, se = st
            base = base0 + i * _CHUNK
            pltpu.sync_copy(nidx.at[pl.ds(base, _CHUNK)], nv)
            pltpu.sync_copy(eidx.at[pl.ds(base, _CHUNK)], ev)
            pltpu.async_copy(ntab.at[nv], nb, sn)
            pltpu.async_copy(mtab.at[nv], mb, sm)
            pltpu.async_copy(etab.at[ev], eb, se)

        def drain(i, st):
            nv, ev, nb, mb, eb, sn, sm, se = st
            base = base0 + i * _CHUNK
            pltpu.make_async_copy(ntab.at[pl.ds(0, _CHUNK)], eb, se).wait()
            pltpu.sync_copy(eb, out_e.at[pl.ds(base, _CHUNK)])
            pltpu.make_async_copy(ntab.at[pl.ds(0, _CHUNK)], nb, sn).wait()
            pltpu.make_async_copy(ntab.at[pl.ds(0, _CHUNK)], mb, sm).wait()
            _vadd_into(nb, mb, _CHUNK)
            pltpu.sync_copy(nb, out_f.at[pl.ds(base, _CHUNK)])

        fire(0, sets[0])

        def body(j2, carry):
            a = 2 * j2
            fire(a + 1, sets[1])
            drain(a, sets[0])

            @pl.when(a + 2 < n_chunks)
            def _():
                fire(a + 2, sets[0])

            drain(a + 1, sets[1])
            return carry

        lax.fori_loop(0, n_chunks // 2, body, 0)

        sbase = wid * seed_per_w
        pltpu.sync_copy(sidx.at[pl.ds(sbase, seed_per_w)], sidx_v)
        cn = pltpu.async_copy(ntab.at[sidx_v], snbuf, sem_na)
        cm = pltpu.async_copy(mtab.at[sidx_v], smbuf, sem_ma)
        cn.wait()
        cm.wait()
        _vadd_into(snbuf, smbuf, seed_per_w)
        pltpu.sync_copy(snbuf, out_sf.at[pl.ds(sbase, seed_per_w)])

    return k(node_table, memory, edge_table, node_idx, seed_idx, edge_idx)


_COSC = (1.0, -19.73920440673828, 64.93911743164062, -85.45014190673828,
         60.16762924194336, -25.967592239379883, 6.52864933013916)
_MAGIC = 1.5 * (2.0 ** 23)
_INV2PI = 0.15915494309189535


def _cos2pi(r):
    k = (r + _MAGIC) - _MAGIC
    f = r - k
    u = f * f
    acc = u * _COSC[6] + _COSC[5]
    for c in _COSC[4::-1]:
        acc = acc * u + c
    return acc


def _typed(x2, oh2, Wn):
    acc = None
    for t in range(NUM_NTYPE):
        y = jnp.dot(x2, Wn[t], preferred_element_type=jnp.float32)
        y = y * oh2[:, t : t + 1]
        acc = y if acc is None else acc + y
    return acc


def _bf(x):
    return x.astype(jnp.bfloat16)


def _fwd_kernel(
    src_s, src_t, nf_s, nf_t,
    ne_s, ne_t, dt_s, dt_t, pen_s, pen_t,
    vm_s, vm_t, nm_s, nm_t, em,
    freq, phase, Wn, Wq, Wk, Wv, f1w, f1b, f2w, f2b, Wd, Hm, Hmt,
    out_ref,
):
    freq_r = freq[...] * _INV2PI
    phase_r = phase[...] * _INV2PI
    Wn_a = Wn[...]
    Wk_a = Wk[...]
    Wv_a = Wv[...]
    Wq_a = Wq[...]
    f1w_a = f1w[...]
    Hm_a = Hm[...]
    Hmt_a = Hmt[...]
    qt = jnp.dot(jnp.cos(phase[...]), Wq_a[D:], preferred_element_type=jnp.float32)

    def local_half(src_ref, nf_ref, ne_ref, dt_ref,
                   pen_ref, vm_ref, nm_ref):
        BS = src_ref.shape[0]
        R = BS * NGH
        feat2 = _bf(nf_ref[...]).reshape(R, D)
        edge2 = _bf(ne_ref[...]).reshape(R, D)
        targ = jnp.dot(dt_ref[...], freq_r, preferred_element_type=jnp.float32)
        temb2 = _bf(_cos2pi(targ + phase_r))
        vm2 = vm_ref[...].reshape(R, NUM_NTYPE)

        trans2 = _bf(_typed(feat2, vm2, _bf(Wn_a)))
        Wk_b = _bf(Wk_a)
        Wv_b = _bf(Wv_a)
        k2 = (
            jnp.dot(trans2, Wk_b[:D], preferred_element_type=jnp.float32)
            + jnp.dot(edge2, Wk_b[D : 2 * D], preferred_element_type=jnp.float32)
            + jnp.dot(temb2, Wk_b[2 * D :], preferred_element_type=jnp.float32)
        )
        v2 = (
            jnp.dot(trans2, Wv_b[:D], preferred_element_type=jnp.float32)
            + jnp.dot(edge2, Wv_b[D : 2 * D], preferred_element_type=jnp.float32)
            + jnp.dot(temb2, Wv_b[2 * D :], preferred_element_type=jnp.float32)
        )

        src_f = src_ref[...]
        trans_s = _typed(src_f, nm_ref[...], Wn_a)
        q = jnp.dot(trans_s, Wq_a[:D], preferred_element_type=jnp.float32) + qt

        q3 = jax.lax.broadcast_in_dim(q, (BS, NGH, D), (0, 2))
        qp3 = k2.reshape(BS, NGH, D) * q3
        s2 = jnp.dot(qp3.reshape(R, D), Hm_a, preferred_element_type=jnp.float32)
        s3 = s2.reshape(BS, NGH, 8) + pen_ref[...].reshape(BS, NGH, 1)
        e = jnp.exp(s3)
        z = jnp.sum(e, axis=1, keepdims=True)
        a3 = e / z
        A2 = jnp.dot(a3.reshape(R, 8), Hmt_a, preferred_element_type=jnp.float32)
        w3 = (A2 * v2).reshape(BS, NGH, D)
        ao = jnp.sum(w3, axis=1)
        h1 = jax.nn.relu(
            jnp.dot(ao, f1w_a[:D], preferred_element_type=jnp.float32)
            + jnp.dot(src_f, f1w_a[D:], preferred_element_type=jnp.float32)
            + f1b[...]
        )
        return jnp.dot(h1, f2w[...], preferred_element_type=jnp.float32) + f2b[...]

    local_s = local_half(src_s, nf_s, ne_s, dt_s, pen_s, vm_s, nm_s)
    local_t = local_half(src_t, nf_t, ne_t, dt_t, pen_t, vm_t, nm_t)

    proj = _typed(local_t, em[...], Wd[...])
    s = jnp.sum(local_s * proj, axis=1, keepdims=True)
    out_ref[0] = 1.0 / (1.0 + jnp.exp(-s))


def _tc_forward(src_feat, ngh_feat, ngh_edge, dt, pen, vtype_oh,
                ntype_oh, etype_oh, basis_freq, phase, W_ntype, Wq, Wk, Wv,
                fc1_w, fc1_b, fc2_w, fc2_b, W_dec):
    B = etype_oh.shape[0]
    BS = _BS
    G = B // BS

    hm = np.zeros((D, 8), np.float32)
    for h in range(N_HEAD):
        hm[h * DH : (h + 1) * DH, h] = 1.0
    Hm = jnp.asarray(hm / math.sqrt(DH))
    Hmt = jnp.asarray(hm.T.copy())

    def half_spec(shape, off):
        nd = len(shape)
        blk = (shape[0] // (2 * G),) + shape[1:]
        return pl.BlockSpec(blk, lambda i, o=off: (o + i,) + (0,) * (nd - 1))

    def full_spec(shape):
        nd = len(shape)
        return pl.BlockSpec(shape, lambda i: (0,) * nd)

    in_specs = []
    args = []

    def add_pair(x):
        args.extend([x, x])
        in_specs.extend([half_spec(x.shape, 0), half_spec(x.shape, G)])

    add_pair(src_feat)
    add_pair(ngh_feat)
    add_pair(ngh_edge)
    add_pair(dt)
    add_pair(pen)
    add_pair(vtype_oh)
    add_pair(ntype_oh)
    args.append(etype_oh)
    in_specs.append(pl.BlockSpec((BS, NUM_ETYPE), lambda i: (i, 0)))

    for w in (basis_freq.reshape(1, T_DIM), phase.reshape(1, T_DIM), W_ntype,
              Wq, Wk, Wv, fc1_w, fc1_b.reshape(1, D), fc2_w,
              fc2_b.reshape(1, D), W_dec, Hm, Hmt):
        args.append(w)
        in_specs.append(full_spec(w.shape))

    out = pl.pallas_call(
        _fwd_kernel,
        grid=(G,),
        in_specs=in_specs,
        out_specs=pl.BlockSpec((1, BS, 1), lambda i: (i, 0, 0)),
        out_shape=jax.ShapeDtypeStruct((G, BS, 1), jnp.float32),
    )(*args)
    return out.reshape(B)


def kernel(src_idx_l, tgt_idx_l, cut_time_l, src_utype_l, tgt_utype_l, etype_l,
           ngh_node, ngh_eidx, ngh_t, ngh_etype, ngh_vtype,
           node_table, edge_table, memory, basis_freq, phase,
           W_ntype, Wq, Wk, Wv, fc1_w, fc1_b, fc2_w, fc2_b, W_dec):
    n = src_idx_l.shape[0]
    nodes = jnp.concatenate([src_idx_l, tgt_idx_l])
    times = jnp.concatenate([cut_time_l, cut_time_l])
    ntypes = jnp.concatenate([src_utype_l, tgt_utype_l])

    dt = (times[:, None] - ngh_t).reshape(-1, 1)
    pen = jnp.where(ngh_node == 0, -30.0, 0.0).astype(jnp.float32).reshape(-1, 1)
    vtype_oh = jax.nn.one_hot(ngh_vtype, NUM_NTYPE, dtype=jnp.bfloat16)
    ntype_oh = jax.nn.one_hot(ntypes, NUM_NTYPE, dtype=jnp.float32)
    etype_oh = jax.nn.one_hot(etype_l, NUM_ETYPE, dtype=jnp.float32)

    out_f, out_e, out_sf = _sc_gather(
        node_table, memory, edge_table,
        ngh_node.reshape(-1).astype(jnp.int32),
        nodes.astype(jnp.int32),
        ngh_eidx.reshape(-1).astype(jnp.int32))
    n2b = 2 * n
    return _tc_forward(
        out_sf,
        out_f.reshape(n2b, NGH, D), out_e.reshape(n2b, NGH, D),
        dt, pen, vtype_oh, ntype_oh, etype_oh, basis_freq, phase,
        W_ntype, Wq, Wk, Wv, fc1_w, fc1_b, fc2_w, fc2_b, W_dec)


# speedup vs baseline: 1.1384x; 1.0369x over previous
import functools
import math

import jax
import jax.numpy as jnp
import numpy as np
from jax import lax
from jax.experimental import pallas as pl
from jax.experimental.pallas import tpu as pltpu
from jax.experimental.pallas import tpu_sc as plsc

D = 128
T_DIM = 128
NGH = 80
N_HEAD = 4
DH = D // N_HEAD
NUM_NTYPE = 3
NUM_ETYPE = 3

_BS = 64

_NW = 32
_CHUNK = 128


def _sc_gather(node_table, memory, edge_table, node_idx, seed_idx, edge_idx):
    NR = edge_idx.shape[0]
    NS = seed_idx.shape[0]
    per_w = NR // _NW
    n_chunks = per_w // _CHUNK
    seed_per_w = NS // _NW

    mesh = plsc.VectorSubcoreMesh(core_axis_name="c", subcore_axis_name="s")

    def _vadd_into(dst, src, rows):
        def rowfn(r, carry):
            for c in range(D // 16):
                sl = pl.ds(c * 16, 16)
                dst[r, sl] = dst[r, sl] + src[r, sl]
            return carry
        lax.fori_loop(0, rows, rowfn, 0)

    @functools.partial(
        pl.kernel,
        out_type=(
            jax.ShapeDtypeStruct((NR, D), jnp.float32),
            jax.ShapeDtypeStruct((NR, D), jnp.float32),
            jax.ShapeDtypeStruct((NS, D), jnp.float32),
        ),
        mesh=mesh,
        scratch_types=[
            pltpu.VMEM((_CHUNK,), jnp.int32),
            pltpu.VMEM((_CHUNK,), jnp.int32),
            pltpu.VMEM((_CHUNK,), jnp.int32),
            pltpu.VMEM((_CHUNK,), jnp.int32),
            pltpu.VMEM((_CHUNK, D), jnp.float32),
            pltpu.VMEM((_CHUNK, D), jnp.float32),
            pltpu.VMEM((_CHUNK, D), jnp.float32),
            pltpu.VMEM((_CHUNK, D), jnp.float32),
            pltpu.VMEM((_CHUNK, D), jnp.float32),
            pltpu.VMEM((_CHUNK, D), jnp.float32),
            pltpu.VMEM((seed_per_w,), jnp.int32),
            pltpu.VMEM((seed_per_w, D), jnp.float32),
            pltpu.VMEM((seed_per_w, D), jnp.float32),
            pltpu.SemaphoreType.DMA,
            pltpu.SemaphoreType.DMA,
            pltpu.SemaphoreType.DMA,
            pltpu.SemaphoreType.DMA,
            pltpu.SemaphoreType.DMA,
            pltpu.SemaphoreType.DMA,
        ],
    )
    def k(ntab, mtab, etab, nidx, sidx, eidx, out_f, out_e, out_sf,
          nidx_a, eidx_a, nidx_b, eidx_b,
          nbuf_a, mbuf_a, ebuf_a, nbuf_b, mbuf_b, ebuf_b,
          sidx_v, snbuf, smbuf,
          sem_na, sem_ma, sem_ea, sem_nb, sem_mb, sem_eb):
        wid = lax.axis_index("s") * 2 + lax.axis_index("c")
        base0 = wid * per_w
        sets = (
            (nidx_a, eidx_a, nbuf_a, mbuf_a, ebuf_a, sem_na, sem_ma, sem_ea),
            (nidx_b, eidx_b, nbuf_b, mbuf_b, ebuf_b, sem_nb, sem_mb, sem_eb),
        )

        def fire(i, st):
            nv, ev, nb, mb, eb, sn, sm, se = st
            base = base0 + i * _CHUNK
            pltpu.sync_copy(nidx.at[pl.ds(base, _CHUNK)], nv)
            pltpu.sync_copy(eidx.at[pl.ds(base, _CHUNK)], ev)
            pltpu.async_copy(ntab.at[nv], nb, sn)
            pltpu.async_copy(mtab.at[nv], mb, sm)
            pltpu.async_copy(etab.at[ev], eb, se)

        def drain(i, st):
            nv, ev, nb, mb, eb, sn, sm, se = st
            base = base0 + i * _CHUNK
            pltpu.make_async_copy(ntab.at[pl.ds(0, _CHUNK)], eb, se).wait()
            pltpu.sync_copy(eb, out_e.at[pl.ds(base, _CHUNK)])
            pltpu.make_async_copy(ntab.at[pl.ds(0, _CHUNK)], nb, sn).wait()
            pltpu.make_async_copy(ntab.at[pl.ds(0, _CHUNK)], mb, sm).wait()
            _vadd_into(nb, mb, _CHUNK)
            pltpu.sync_copy(nb, out_f.at[pl.ds(base, _CHUNK)])

        fire(0, sets[0])

        def body(j2, carry):
            a = 2 * j2
            fire(a + 1, sets[1])
            drain(a, sets[0])

            @pl.when(a + 2 < n_chunks)
            def _():
                fire(a + 2, sets[0])

            drain(a + 1, sets[1])
            return carry

        lax.fori_loop(0, n_chunks // 2, body, 0)

        sbase = wid * seed_per_w
        pltpu.sync_copy(sidx.at[pl.ds(sbase, seed_per_w)], sidx_v)
        cn = pltpu.async_copy(ntab.at[sidx_v], snbuf, sem_na)
        cm = pltpu.async_copy(mtab.at[sidx_v], smbuf, sem_ma)
        cn.wait()
        cm.wait()
        _vadd_into(snbuf, smbuf, seed_per_w)
        pltpu.sync_copy(snbuf, out_sf.at[pl.ds(sbase, seed_per_w)])

    return k(node_table, memory, edge_table, node_idx, seed_idx, edge_idx)


_COSC = (1.0, -19.73920440673828, 64.93911743164062, -85.45014190673828,
         60.16762924194336, -25.967592239379883, 6.52864933013916)
_MAGIC = 1.5 * (2.0 ** 23)
_INV2PI = 0.15915494309189535


def _cos2pi(r):
    k = (r + _MAGIC) - _MAGIC
    f = r - k
    u = f * f
    acc = u * _COSC[6] + _COSC[5]
    for c in _COSC[4::-1]:
        acc = acc * u + c
    return acc


def _typed(x2, oh2, Wn):
    acc = None
    for t in range(NUM_NTYPE):
        y = jnp.dot(x2, Wn[t], preferred_element_type=jnp.float32)
        y = y * oh2[:, t : t + 1]
        acc = y if acc is None else acc + y
    return acc


def _bf(x):
    return x.astype(jnp.bfloat16)


def _fwd_kernel(
    src_s, src_t, nf_s, nf_t,
    ne_s, ne_t, dt_s, dt_t, pen_s, pen_t,
    vm_s, vm_t, nm_s, nm_t, em,
    freq, phase, Wn, Wq, Wk, Wv, f1w, f1b, f2w, f2b, Wd, Hm, Hmt,
    out_ref,
):
    freq_r = freq[...] * _INV2PI
    phase_r = phase[...] * _INV2PI
    Wn_a = Wn[...]
    Wk_a = Wk[...]
    Wv_a = Wv[...]
    Wq_a = Wq[...]
    f1w_a = f1w[...]
    Hm_a = Hm[...]
    Hmt_a = Hmt[...]
    qt = jnp.dot(jnp.cos(phase[...]), Wq_a[D:], preferred_element_type=jnp.float32)

    def local_half(src_ref, nf_ref, ne_ref, dt_ref,
                   pen_ref, vm_ref, nm_ref):
        BS = src_ref.shape[0]
        R = BS * NGH
        feat2 = _bf(nf_ref[...]).reshape(R, D)
        edge2 = _bf(ne_ref[...]).reshape(R, D)
        targ = jnp.dot(dt_ref[...], freq_r, preferred_element_type=jnp.float32)
        temb2 = _bf(_cos2pi(targ + phase_r))
        vm2 = vm_ref[...].reshape(R, NUM_NTYPE)

        trans2 = _bf(_typed(feat2, vm2, _bf(Wn_a)))
        Wk_b = _bf(Wk_a)
        Wv_b = _bf(Wv_a)
        k2 = (
            jnp.dot(trans2, Wk_b[:D], preferred_element_type=jnp.float32)
            + jnp.dot(edge2, Wk_b[D : 2 * D], preferred_element_type=jnp.float32)
            + jnp.dot(temb2, Wk_b[2 * D :], preferred_element_type=jnp.float32)
        )
        v2 = (
            jnp.dot(trans2, Wv_b[:D], preferred_element_type=jnp.float32)
            + jnp.dot(edge2, Wv_b[D : 2 * D], preferred_element_type=jnp.float32)
            + jnp.dot(temb2, Wv_b[2 * D :], preferred_element_type=jnp.float32)
        )

        src_f = src_ref[...]
        trans_s = _typed(src_f, nm_ref[...], Wn_a)
        q = jnp.dot(trans_s, Wq_a[:D], preferred_element_type=jnp.float32) + qt

        q3 = jax.lax.broadcast_in_dim(q, (BS, NGH, D), (0, 2))
        qp3 = k2.reshape(BS, NGH, D) * q3
        s2 = jnp.dot(qp3.reshape(R, D), Hm_a, preferred_element_type=jnp.float32)
        s3 = s2.reshape(BS, NGH, 8) + pen_ref[...].reshape(BS, NGH, 1)
        e = jnp.exp(s3)
        z = jnp.sum(e, axis=1, keepdims=True)
        a3 = e / z
        A2 = jnp.dot(a3.reshape(R, 8), Hmt_a, preferred_element_type=jnp.float32)
        w3 = (A2 * v2).reshape(BS, NGH, D)
        ao = jnp.sum(w3, axis=1)
        h1 = jax.nn.relu(
            jnp.dot(ao, f1w_a[:D], preferred_element_type=jnp.float32)
            + jnp.dot(src_f, f1w_a[D:], preferred_element_type=jnp.float32)
            + f1b[...]
        )
        return jnp.dot(h1, f2w[...], preferred_element_type=jnp.float32) + f2b[...]

    local_s = local_half(src_s, nf_s, ne_s, dt_s, pen_s, vm_s, nm_s)
    local_t = local_half(src_t, nf_t, ne_t, dt_t, pen_t, vm_t, nm_t)

    proj = _typed(local_t, em[...], Wd[...])
    s = jnp.sum(local_s * proj, axis=1, keepdims=True)
    out_ref[0] = 1.0 / (1.0 + jnp.exp(-s))


def _tc_forward(src_feat, ngh_feat, ngh_edge, dt, pen, vtype_oh,
                ntype_oh, etype_oh, basis_freq, phase, W_ntype, Wq, Wk, Wv,
                fc1_w, fc1_b, fc2_w, fc2_b, W_dec):
    B = etype_oh.shape[0]
    BS = _BS
    G = B // BS

    hm = np.zeros((D, 8), np.float32)
    for h in range(N_HEAD):
        hm[h * DH : (h + 1) * DH, h] = 1.0
    Hm = jnp.asarray(hm / math.sqrt(DH))
    Hmt = jnp.asarray(hm.T.copy())

    def half_spec(shape, off):
        nd = len(shape)
        blk = (shape[0] // (2 * G),) + shape[1:]
        return pl.BlockSpec(blk, lambda i, o=off: (o + i,) + (0,) * (nd - 1))

    def full_spec(shape):
        nd = len(shape)
        return pl.BlockSpec(shape, lambda i: (0,) * nd)

    in_specs = []
    args = []

    def add_pair(x):
        args.extend([x, x])
        in_specs.extend([half_spec(x.shape, 0), half_spec(x.shape, G)])

    add_pair(src_feat)
    add_pair(ngh_feat)
    add_pair(ngh_edge)
    add_pair(dt)
    add_pair(pen)
    add_pair(vtype_oh)
    add_pair(ntype_oh)
    args.append(etype_oh)
    in_specs.append(pl.BlockSpec((BS, NUM_ETYPE), lambda i: (i, 0)))

    for w in (basis_freq.reshape(1, T_DIM), phase.reshape(1, T_DIM), W_ntype,
              Wq, Wk, Wv, fc1_w, fc1_b.reshape(1, D), fc2_w,
              fc2_b.reshape(1, D), W_dec, Hm, Hmt):
        args.append(w)
        in_specs.append(full_spec(w.shape))

    out = pl.pallas_call(
        _fwd_kernel,
        grid=(G,),
        in_specs=in_specs,
        out_specs=pl.BlockSpec((1, BS, 1), lambda i: (i, 0, 0)),
        out_shape=jax.ShapeDtypeStruct((G, BS, 1), jnp.float32),
    )(*args)
    return out.reshape(B)


def kernel(src_idx_l, tgt_idx_l, cut_time_l, src_utype_l, tgt_utype_l, etype_l,
           ngh_node, ngh_eidx, ngh_t, ngh_etype, ngh_vtype,
           node_table, edge_table, memory, basis_freq, phase,
           W_ntype, Wq, Wk, Wv, fc1_w, fc1_b, fc2_w, fc2_b, W_dec):
    n = src_idx_l.shape[0]
    nodes = jnp.concatenate([src_idx_l, tgt_idx_l])
    times = jnp.concatenate([cut_time_l, cut_time_l])
    ntypes = jnp.concatenate([src_utype_l, tgt_utype_l])

    dt = (times[:, None] - ngh_t).reshape(-1, 1)
    pen = jnp.where(ngh_node == 0, -30.0, 0.0).astype(jnp.float32).reshape(-1, 1)
    vtype_oh = jax.nn.one_hot(ngh_vtype, NUM_NTYPE, dtype=jnp.bfloat16)
    ntype_oh = jax.nn.one_hot(ntypes, NUM_NTYPE, dtype=jnp.float32)
    etype_oh = jax.nn.one_hot(etype_l, NUM_ETYPE, dtype=jnp.float32)

    out_f, out_e, out_sf = _sc_gather(
        node_table, memory, edge_table,
        ngh_node.reshape(-1).astype(jnp.int32),
        nodes.astype(jnp.int32),
        ngh_eidx.reshape(-1).astype(jnp.int32))
    n2b = 2 * n
    return _tc_forward(
        out_sf,
        out_f.reshape(n2b, NGH, D), out_e.reshape(n2b, NGH, D),
        dt, pen, vtype_oh, ntype_oh, etype_oh, basis_freq, phase,
        W_ntype, Wq, Wk, Wv, fc1_w, fc1_b, fc2_w, fc2_b, W_dec)
